# baseline (speedup 1.0000x reference)
import functools
import math

import jax
import jax.numpy as jnp
from jax import lax
from jax.experimental import pallas as pl
from jax.experimental.pallas import tpu as pltpu
from jax.experimental.pallas import tpu_sc as plsc

_NUM_ROWS = 1024
_RPB = 16
_NUM_BLOCKS = 64
_D = 4096
_SCALE = math.sqrt(_NUM_ROWS / _RPB)
_LANES = 16

_NW = 32
_ROWS_PER_W = _NUM_ROWS // _NW
_CHUNK_ROWS = 8
_NCHUNKS = _ROWS_PER_W // _CHUNK_ROWS
_NBUF = 3


def _make_sc_kernel():
    mesh = plsc.VectorSubcoreMesh(core_axis_name="c", subcore_axis_name="s")

    @functools.partial(
        pl.kernel,
        mesh=mesh,
        out_type=jax.ShapeDtypeStruct((_NUM_BLOCKS, _RPB, _D), jnp.float32),
        scratch_types=(
            [pltpu.VMEM((_CHUNK_ROWS, _D), jnp.float32)] * _NBUF
            + [pltpu.SemaphoreType.DMA] * (2 * _NBUF)
        ),
    )
    def sc_kernel(in_hbm, out_hbm, b0, b1, b2, si0, si1, si2, so0, so1, so2):
        wid = lax.axis_index("s") * 2 + lax.axis_index("c")
        row0 = wid * _ROWS_PER_W
        bufs = (b0, b1, b2)
        in_sems = (si0, si1, si2)
        out_sems = (so0, so1, so2)
        has_zero = (wid % 4) == 0

        def in_copy(g):
            src = in_hbm.at[pl.ds(row0 + g * _CHUNK_ROWS, _CHUNK_ROWS), :]
            return pltpu.make_async_copy(src, bufs[g % _NBUF], in_sems[g % _NBUF])

        def start_in(g):
            if g < 2:
                @pl.when(jnp.logical_not(has_zero))
                def _():
                    in_copy(g).start()
            else:
                in_copy(g).start()

        def wait_in(g):
            if g < 2:
                @pl.when(jnp.logical_not(has_zero))
                def _():
                    in_copy(g).wait()
            else:
                in_copy(g).wait()

        def start_out(g):
            block = wid * 2 + (g // 2)
            dst = out_hbm.at[block, pl.ds((g % 2) * _CHUNK_ROWS, _CHUNK_ROWS), :]
            return pltpu.async_copy(bufs[g % _NBUF], dst, out_sems[g % _NBUF])

        for g in range(_NBUF):
            start_in(g)
        out_handles = {}
        waited_out = set()
        for g in range(_NCHUNKS):
            nxt = g + _NBUF - 1
            if nxt >= _NBUF and nxt < _NCHUNKS:
                out_handles[nxt - _NBUF].wait()
                waited_out.add(nxt - _NBUF)
                start_in(nxt)
            wait_in(g)
            buf = bufs[g % _NBUF]
            if g < 2:
                @pl.when(has_zero)
                def _(buf=buf):
                    zeros = jnp.zeros((_LANES,), jnp.float32)

                    @plsc.parallel_loop(0, _D, step=_LANES, unroll=8)
                    def zero_body(i, buf=buf, zeros=zeros):
                        for r in range(_CHUNK_ROWS):
                            buf[r, pl.ds(i, _LANES)] = zeros

                @pl.when(jnp.logical_not(has_zero))
                def _(buf=buf):
                    @plsc.parallel_loop(0, _D, step=_LANES, unroll=8)
                    def scale_body(i, buf=buf):
                        for r in range(_CHUNK_ROWS):
                            sl = pl.ds(i, _LANES)
                            buf[r, sl] = buf[r, sl] * _SCALE
            else:
                @plsc.parallel_loop(0, _D, step=_LANES, unroll=8)
                def scale_body(i, buf=buf):
                    for r in range(_CHUNK_ROWS):
                        sl = pl.ds(i, _LANES)
                        buf[r, sl] = buf[r, sl] * _SCALE

            out_handles[g] = start_out(g)
        for g in range(_NCHUNKS):
            if g not in waited_out:
                out_handles[g].wait()

    return sc_kernel


_sc_kernel = _make_sc_kernel()


@jax.jit
def kernel(inputs):
    return _sc_kernel(inputs)


# speedup vs baseline: 1.0570x; 1.0570x over previous
import functools
import math

import jax
import jax.numpy as jnp
from jax import lax
from jax.experimental import pallas as pl
from jax.experimental.pallas import tpu as pltpu
from jax.experimental.pallas import tpu_sc as plsc

_NUM_ROWS = 1024
_RPB = 16
_NUM_BLOCKS = 64
_D = 4096
_SCALE = math.sqrt(_NUM_ROWS / _RPB)
_LANES = 16

_NW = 32
_ROWS_PER_W = _NUM_ROWS // _NW
_CHUNK_ROWS = 4
_NCHUNKS = _ROWS_PER_W // _CHUNK_ROWS
_CPB = _RPB // _CHUNK_ROWS
_NBUF = 6
_UNROLL = 8


def _make_sc_kernel():
    mesh = plsc.VectorSubcoreMesh(core_axis_name="c", subcore_axis_name="s")

    @functools.partial(
        pl.kernel,
        mesh=mesh,
        out_type=jax.ShapeDtypeStruct((_NUM_BLOCKS, _RPB, _D), jnp.float32),
        scratch_types=(
            [pltpu.VMEM((_CHUNK_ROWS, _D), jnp.float32)] * _NBUF
            + [pltpu.SemaphoreType.DMA] * (2 * _NBUF)
        ),
    )
    def sc_kernel(in_hbm, out_hbm, *scratch):
        bufs = scratch[:_NBUF]
        in_sems = scratch[_NBUF:2 * _NBUF]
        out_sems = scratch[2 * _NBUF:]
        wid = lax.axis_index("s") * 2 + lax.axis_index("c")
        row0 = wid * _ROWS_PER_W
        has_zero = (wid % 4) == 0
        not_zero = jnp.logical_not(has_zero)

        def in_copy(g):
            src = in_hbm.at[pl.ds(row0 + g * _CHUNK_ROWS, _CHUNK_ROWS), :]
            return pltpu.make_async_copy(src, bufs[g % _NBUF], in_sems[g % _NBUF])

        def start_in(g):
            if g < _CPB:
                @pl.when(not_zero)
                def _():
                    in_copy(g).start()
            else:
                in_copy(g).start()

        def wait_in(g):
            if g < _CPB:
                @pl.when(not_zero)
                def _():
                    in_copy(g).wait()
            else:
                in_copy(g).wait()

        def start_out(g):
            block = wid * 2 + (g // _CPB)
            dst = out_hbm.at[block, pl.ds((g % _CPB) * _CHUNK_ROWS, _CHUNK_ROWS), :]
            return pltpu.async_copy(bufs[g % _NBUF], dst, out_sems[g % _NBUF])

        def scale_chunk(buf):
            @plsc.parallel_loop(0, _D, step=_LANES, unroll=_UNROLL)
            def scale_body(i, buf=buf):
                for r in range(_CHUNK_ROWS):
                    sl = pl.ds(i, _LANES)
                    buf[r, sl] = buf[r, sl] * _SCALE

        def zero_chunk(buf):
            zeros = jnp.zeros((_LANES,), jnp.float32)

            @plsc.parallel_loop(0, _D, step=_LANES, unroll=_UNROLL)
            def zero_body(i, buf=buf, zeros=zeros):
                for r in range(_CHUNK_ROWS):
                    buf[r, pl.ds(i, _LANES)] = zeros

        for g in range(_NBUF):
            start_in(g)
        out_handles = {}
        waited_out = set()
        for g in range(_NCHUNKS):
            nxt = g + _NBUF - 1
            if nxt >= _NBUF and nxt < _NCHUNKS:
                out_handles[nxt - _NBUF].wait()
                waited_out.add(nxt - _NBUF)
                start_in(nxt)
            wait_in(g)
            buf = bufs[g % _NBUF]
            if g < _CPB:
                @pl.when(has_zero)
                def _(buf=buf):
                    zero_chunk(buf)

                @pl.when(not_zero)
                def _(buf=buf):
                    scale_chunk(buf)
            else:
                scale_chunk(buf)

            out_handles[g] = start_out(g)
        for g in range(_NCHUNKS):
            if g not in waited_out:
                out_handles[g].wait()

    return sc_kernel


_sc_kernel = _make_sc_kernel()


@jax.jit
def kernel(inputs):
    return _sc_kernel(inputs)


# speedup vs baseline: 1.0743x; 1.0163x over previous
import functools
import math

import jax
import jax.numpy as jnp
from jax import lax
from jax.experimental import pallas as pl
from jax.experimental.pallas import tpu as pltpu
from jax.experimental.pallas import tpu_sc as plsc

_NUM_ROWS = 1024
_RPB = 16
_NUM_BLOCKS = 64
_D = 4096
_SCALE = math.sqrt(_NUM_ROWS / _RPB)
_LANES = 16

_NW = 32
_ROWS_PER_W = _NUM_ROWS // _NW
_CHUNK_ROWS = 2
_NCHUNKS = _ROWS_PER_W // _CHUNK_ROWS
_CPB = _RPB // _CHUNK_ROWS
_NBUF = 12
_UNROLL = 8


def _make_sc_kernel():
    mesh = plsc.VectorSubcoreMesh(core_axis_name="c", subcore_axis_name="s")

    @functools.partial(
        pl.kernel,
        mesh=mesh,
        out_type=jax.ShapeDtypeStruct((_NUM_BLOCKS, _RPB, _D), jnp.float32),
        scratch_types=(
            [pltpu.VMEM((_CHUNK_ROWS, _D), jnp.float32)] * _NBUF
            + [pltpu.SemaphoreType.DMA] * (2 * _NBUF)
        ),
    )
    def sc_kernel(in_hbm, out_hbm, *scratch):
        bufs = scratch[:_NBUF]
        in_sems = scratch[_NBUF:2 * _NBUF]
        out_sems = scratch[2 * _NBUF:]
        wid = lax.axis_index("s") * 2 + lax.axis_index("c")
        row0 = wid * _ROWS_PER_W
        has_zero = (wid % 4) == 0
        not_zero = jnp.logical_not(has_zero)

        def in_copy(g):
            src = in_hbm.at[pl.ds(row0 + g * _CHUNK_ROWS, _CHUNK_ROWS), :]
            return pltpu.make_async_copy(src, bufs[g % _NBUF], in_sems[g % _NBUF])

        def start_in(g):
            if g < _CPB:
                @pl.when(not_zero)
                def _():
                    in_copy(g).start()
            else:
                in_copy(g).start()

        def wait_in(g):
            if g < _CPB:
                @pl.when(not_zero)
                def _():
                    in_copy(g).wait()
            else:
                in_copy(g).wait()

        def start_out(g):
            block = wid * 2 + (g // _CPB)
            dst = out_hbm.at[block, pl.ds((g % _CPB) * _CHUNK_ROWS, _CHUNK_ROWS), :]
            return pltpu.async_copy(bufs[g % _NBUF], dst, out_sems[g % _NBUF])

        def scale_chunk(buf):
            @plsc.parallel_loop(0, _D, step=_LANES, unroll=_UNROLL)
            def scale_body(i, buf=buf):
                for r in range(_CHUNK_ROWS):
                    sl = pl.ds(i, _LANES)
                    buf[r, sl] = buf[r, sl] * _SCALE

        def zero_chunk(buf):
            zeros = jnp.zeros((_LANES,), jnp.float32)

            @plsc.parallel_loop(0, _D, step=_LANES, unroll=_UNROLL)
            def zero_body(i, buf=buf, zeros=zeros):
                for r in range(_CHUNK_ROWS):
                    buf[r, pl.ds(i, _LANES)] = zeros

        for g in range(_NBUF):
            start_in(g)
        out_handles = {}
        waited_out = set()
        for g in range(_NCHUNKS):
            nxt = g + _NBUF - 1
            if nxt >= _NBUF and nxt < _NCHUNKS:
                out_handles[nxt - _NBUF].wait()
                waited_out.add(nxt - _NBUF)
                start_in(nxt)
            wait_in(g)
            buf = bufs[g % _NBUF]
            if g < _CPB:
                @pl.when(has_zero)
                def _(buf=buf):
                    zero_chunk(buf)

                @pl.when(not_zero)
                def _(buf=buf):
                    scale_chunk(buf)
            else:
                scale_chunk(buf)

            out_handles[g] = start_out(g)
        for g in range(_NCHUNKS):
            if g not in waited_out:
                out_handles[g].wait()

    return sc_kernel


_sc_kernel = _make_sc_kernel()


@jax.jit
def kernel(inputs):
    return _sc_kernel(inputs)
